# baseline (speedup 1.0000x reference)
import functools

import jax
import jax.numpy as jnp
from jax import lax
from jax.experimental import pallas as pl
from jax.experimental.pallas import tpu as pltpu
from jax.experimental.pallas import tpu_sc as plsc

NUM_CORES = 2
NUM_SUBCORES = 16
LANES = 16
NW = NUM_CORES * NUM_SUBCORES

EMBED_DIM = 64
BATCH = 16384
B_PER_W = BATCH // NW


def _sc_kernel(s_hbm, p_hbm, o_hbm, ent_hbm, rel_hbm, out_hbm,
               s_idx, p_idx, o_idx, s_rows, p_rows, o_rows, out_v,
               sem_s, sem_p, sem_o):
    wid = lax.axis_index("s") * NUM_CORES + lax.axis_index("c")
    base = wid * B_PER_W

    pltpu.sync_copy(s_hbm.at[pl.ds(base, B_PER_W)], s_idx)
    pltpu.sync_copy(p_hbm.at[pl.ds(base, B_PER_W)], p_idx)
    pltpu.sync_copy(o_hbm.at[pl.ds(base, B_PER_W)], o_idx)

    cp_s = pltpu.async_copy(ent_hbm.at[s_idx], s_rows, sem_s)
    cp_p = pltpu.async_copy(rel_hbm.at[p_idx], p_rows, sem_p)
    cp_o = pltpu.async_copy(ent_hbm.at[o_idx], o_rows, sem_o)
    cp_s.wait()
    cp_p.wait()
    cp_o.wait()

    def body(g, _):
        rows = lax.iota(jnp.int32, LANES) + g * LANES
        acc = jnp.zeros((LANES,), jnp.float32)
        for d in range(EMBED_DIM):
            cols = jnp.full((LANES,), d, jnp.int32)
            acc = acc + (plsc.load_gather(s_rows, [rows, cols])
                         * plsc.load_gather(p_rows, [rows, cols])
                         * plsc.load_gather(o_rows, [rows, cols]))
        out_v[pl.ds(g * LANES, LANES)] = acc
        return 0

    lax.fori_loop(0, B_PER_W // LANES, body, 0)

    pltpu.sync_copy(out_v, out_hbm.at[pl.ds(base, B_PER_W)])


@jax.jit
def _run(s, p, o, entity_embeddings, relation_embeddings):
    mesh = plsc.VectorSubcoreMesh(core_axis_name="c", subcore_axis_name="s")
    f = functools.partial(
        pl.kernel,
        out_type=jax.ShapeDtypeStruct((BATCH,), jnp.float32),
        mesh=mesh,
        compiler_params=pltpu.CompilerParams(
            needs_layout_passes=False, use_tc_tiling_on_sc=False),
        scratch_types=[
            pltpu.VMEM((B_PER_W,), jnp.int32),
            pltpu.VMEM((B_PER_W,), jnp.int32),
            pltpu.VMEM((B_PER_W,), jnp.int32),
            pltpu.VMEM((B_PER_W, EMBED_DIM), jnp.float32),
            pltpu.VMEM((B_PER_W, EMBED_DIM), jnp.float32),
            pltpu.VMEM((B_PER_W, EMBED_DIM), jnp.float32),
            pltpu.VMEM((B_PER_W,), jnp.float32),
            pltpu.SemaphoreType.DMA,
            pltpu.SemaphoreType.DMA,
            pltpu.SemaphoreType.DMA,
        ],
    )(_sc_kernel)
    return f(s, p, o, entity_embeddings, relation_embeddings)


def kernel(s, p, o, entity_embeddings, relation_embeddings):
    return _run(s.astype(jnp.int32), p.astype(jnp.int32), o.astype(jnp.int32),
                entity_embeddings, relation_embeddings)


# speedup vs baseline: 1.5108x; 1.5108x over previous
import functools

import jax
import jax.numpy as jnp
from jax import lax
from jax.experimental import pallas as pl
from jax.experimental.pallas import tpu as pltpu
from jax.experimental.pallas import tpu_sc as plsc

NUM_CORES = 2
NUM_SUBCORES = 16
LANES = 16
NW = NUM_CORES * NUM_SUBCORES

EMBED_DIM = 64
BATCH = 16384
B_PER_W = BATCH // NW
CHUNK = 32
NCHUNK = B_PER_W // CHUNK
GROUPS = CHUNK // LANES


def _sc_kernel(s_hbm, p_hbm, o_hbm, ent_hbm, rel_hbm, out_hbm,
               s_idx, p_idx, o_idx,
               sg, pg, og, stage, out_v,
               sem_s, sem_p, sem_o):
    wid = lax.axis_index("s") * NUM_CORES + lax.axis_index("c")
    base = wid * B_PER_W

    pltpu.sync_copy(s_hbm.at[pl.ds(base, B_PER_W)], s_idx)
    pltpu.sync_copy(p_hbm.at[pl.ds(base, B_PER_W)], p_idx)
    pltpu.sync_copy(o_hbm.at[pl.ds(base, B_PER_W)], o_idx)

    def chunk_body(c, _):
        cb = c * CHUNK
        copies = []
        for g in range(GROUPS):
            sv = s_idx[pl.ds(cb + g * LANES, LANES)]
            pv = p_idx[pl.ds(cb + g * LANES, LANES)]
            ov = o_idx[pl.ds(cb + g * LANES, LANES)]
            for j in range(LANES):
                row = g * LANES + j
                for idx_vec, tbl, buf, sem in ((sv, ent_hbm, sg, sem_s),
                                               (pv, rel_hbm, pg, sem_p),
                                               (ov, ent_hbm, og, sem_o)):
                    r = idx_vec[j]
                    grp = pl.multiple_of((r >> 3) << 3, 8)
                    copies.append(pltpu.async_copy(
                        tbl.at[pl.ds(grp, 8), :], buf.at[row], sem))
        for cp in copies:
            cp.wait()

        for g in range(GROUPS):
            gsl = pl.ds(cb + g * LANES, LANES)
            sv = s_idx[gsl]
            pv = p_idx[gsl]
            ov = o_idx[gsl]
            for j in range(LANES):
                row = g * LANES + j
                rs = sv[j] & 7
                rp = pv[j] & 7
                ro = ov[j] & 7
                acc = (sg[row, rs, pl.ds(0, LANES)]
                       * pg[row, rp, pl.ds(0, LANES)]
                       * og[row, ro, pl.ds(0, LANES)])
                for d in range(1, EMBED_DIM // LANES):
                    acc = acc + (sg[row, rs, pl.ds(d * LANES, LANES)]
                                 * pg[row, rp, pl.ds(d * LANES, LANES)]
                                 * og[row, ro, pl.ds(d * LANES, LANES)])
                stage[pl.ds(j * LANES, LANES)] = acc
            col = lax.iota(jnp.int32, LANES) * LANES
            out_vec = plsc.load_gather(stage, [col])
            for l in range(1, LANES):
                out_vec = out_vec + plsc.load_gather(stage, [col + l])
            out_v[gsl] = out_vec
        return 0

    lax.fori_loop(0, NCHUNK, chunk_body, 0)

    pltpu.sync_copy(out_v, out_hbm.at[pl.ds(base, B_PER_W)])


@jax.jit
def _run(s, p, o, entity_embeddings, relation_embeddings):
    mesh = plsc.VectorSubcoreMesh(core_axis_name="c", subcore_axis_name="s")
    f = functools.partial(
        pl.kernel,
        out_type=jax.ShapeDtypeStruct((BATCH,), jnp.float32),
        mesh=mesh,
        compiler_params=pltpu.CompilerParams(
            needs_layout_passes=False, use_tc_tiling_on_sc=True),
        scratch_types=[
            pltpu.VMEM((B_PER_W,), jnp.int32),
            pltpu.VMEM((B_PER_W,), jnp.int32),
            pltpu.VMEM((B_PER_W,), jnp.int32),
            pltpu.VMEM((CHUNK, 8, EMBED_DIM), jnp.float32),
            pltpu.VMEM((CHUNK, 8, EMBED_DIM), jnp.float32),
            pltpu.VMEM((CHUNK, 8, EMBED_DIM), jnp.float32),
            pltpu.VMEM((LANES * LANES,), jnp.float32),
            pltpu.VMEM((B_PER_W,), jnp.float32),
            pltpu.SemaphoreType.DMA,
            pltpu.SemaphoreType.DMA,
            pltpu.SemaphoreType.DMA,
        ],
    )(_sc_kernel)
    return f(s, p, o, entity_embeddings, relation_embeddings)


def kernel(s, p, o, entity_embeddings, relation_embeddings):
    return _run(s.astype(jnp.int32), p.astype(jnp.int32), o.astype(jnp.int32),
                entity_embeddings, relation_embeddings)
